# Initial kernel scaffold; baseline (speedup 1.0000x reference)
import jax
import jax.numpy as jnp
from jax.experimental import pallas as pl


def kernel(x, tid_data, diw_data, time_day, time_week):
    raise NotImplementedError("write your pallas kernel here")


# speedup vs baseline: 8.3850x; 8.3850x over previous
import functools

import jax
import jax.numpy as jnp
from jax import lax
from jax.experimental import pallas as pl
from jax.experimental.pallas import tpu as pltpu
from jax.experimental.pallas import tpu_sc as plsc

TIME = 288
WEEK = 7
FEAT = 64
B = 64
N = 16384
K = TIME * WEEK
L = 16

NCORES = 2
NSUB = 16
NW = NCORES * NSUB
B_PER_W = B // NW
ROWS_PER_SUB = FEAT // NSUB
FT = 8
NCHUNK = 1024
NCH = N // NCHUNK
VPC = NCHUNK // L


def _body(tid_hbm, diw_hbm, td_hbm, tw_hbm, out_hbm,
          td_v, tw_v, tbl_tile, idx_v, tid_v, diw_v, out_v,
          sem0, sem1, tbl_sh):
    c = lax.axis_index("c")
    s = lax.axis_index("s")
    wid = s * NCORES + c

    pltpu.sync_copy(td_hbm, td_v)
    pltpu.sync_copy(tw_hbm, tw_v)

    lanes = lax.iota(jnp.int32, L)

    def build_chunk(ch, _):
        kv = ch * L + lanes
        iv = lax.div(kv, WEEK)
        jv = kv - iv * WEEK
        for r in range(ROWS_PER_SUB):
            f = s * ROWS_PER_SUB + r
            tdv = plsc.load_gather(td_v, [iv * FEAT + f])
            twv = plsc.load_gather(tw_v, [jv * FEAT + f])
            tbl_tile[pl.ds(r * K + ch * L, L)] = tdv + twv
        return 0

    lax.fori_loop(0, K // L, build_chunk, 0)
    pltpu.sync_copy(tbl_tile.at[pl.ds(0, ROWS_PER_SUB * K)],
                    tbl_sh.at[pl.ds(s * ROWS_PER_SUB * K, ROWS_PER_SUB * K)])
    plsc.subcore_barrier()

    for bi in range(B_PER_W):
        b = wid * B_PER_W + bi

        pltpu.sync_copy(tid_hbm.at[b], tid_v)
        pltpu.sync_copy(diw_hbm.at[b], diw_v)

        def idx_body(v, _):
            t = tid_v[pl.ds(v * L, L)]
            ti = jnp.clip((t * float(TIME)).astype(jnp.int32), 0, TIME - 1)
            wi = jnp.clip(diw_v[pl.ds(v * L, L)], 0, WEEK - 1)
            idx_v[pl.ds(v * L, L)] = ti * WEEK + wi
            return 0

        lax.fori_loop(0, N // L, idx_body, 0)

        def ft_body(ft, _):
            pltpu.sync_copy(tbl_sh.at[pl.ds(ft * FT * K, FT * K)], tbl_tile)

            def fill(buf, nch):
                base = nch * NCHUNK

                def v_body(v, _):
                    idxv = idx_v[pl.ds(base + v * L, L)]
                    for f8 in range(FT):
                        out_v[buf, f8, pl.ds(v * L, L)] = plsc.load_gather(
                            tbl_tile.at[pl.ds(f8 * K, K)], [idxv])
                    return 0

                lax.fori_loop(0, VPC, v_body, 0)

            def fire(buf, nch, sem):
                return pltpu.async_copy(
                    out_v.at[buf],
                    out_hbm.at[b, pl.ds(ft * FT, FT), pl.ds(nch * NCHUNK, NCHUNK)],
                    sem)

            def drain(sem):
                pltpu.make_async_copy(
                    out_v.at[0],
                    out_hbm.at[b, pl.ds(0, FT), pl.ds(0, NCHUNK)],
                    sem).wait()

            def chunk_body(t, _):
                @pl.when(t > 0)
                def _():
                    drain(sem0)
                fill(0, 2 * t)
                fire(0, 2 * t, sem0)

                @pl.when(t > 0)
                def _():
                    drain(sem1)
                fill(1, 2 * t + 1)
                fire(1, 2 * t + 1, sem1)
                return 0

            lax.fori_loop(0, NCH // 2, chunk_body, 0)
            drain(sem0)
            drain(sem1)
            return 0

        lax.fori_loop(0, FEAT // FT, ft_body, 0)


@jax.jit
def _run(tid_last, diw_last, time_day, time_week):
    mesh = plsc.VectorSubcoreMesh(core_axis_name="c", subcore_axis_name="s")
    f = pl.kernel(
        _body,
        out_type=jax.ShapeDtypeStruct((B, FEAT, N), jnp.float32),
        mesh=mesh,
        compiler_params=pltpu.CompilerParams(needs_layout_passes=False),
        scratch_types=[
            pltpu.VMEM((TIME * FEAT,), jnp.float32),
            pltpu.VMEM((WEEK * FEAT,), jnp.float32),
            pltpu.VMEM((FT * K,), jnp.float32),
            pltpu.VMEM((N,), jnp.int32),
            pltpu.VMEM((N,), jnp.float32),
            pltpu.VMEM((N,), jnp.int32),
            pltpu.VMEM((2, FT, NCHUNK), jnp.float32),
            pltpu.SemaphoreType.DMA,
            pltpu.SemaphoreType.DMA,
            pltpu.VMEM_SHARED((FEAT * K,), jnp.float32),
        ],
    )
    return f(tid_last, diw_last, time_day.reshape(-1), time_week.reshape(-1))


def kernel(x, tid_data, diw_data, time_day, time_week):
    del x
    tid_last = tid_data[:, -1, :]
    diw_last = diw_data[:, -1, :]
    out = _run(tid_last, diw_last, time_day, time_week)
    return out[..., None]
